# baseline (speedup 1.0000x reference)
import functools

import jax
import jax.numpy as jnp
from jax import lax
from jax.experimental import pallas as pl
from jax.experimental.pallas import tpu as pltpu
from jax.experimental.pallas import tpu_sc as plsc

_D = 32
_L = 16
_NC = 2
_NS = 16
_NW = _NC * _NS
_RB = 12
_OB = 64
_CAP = 16
_BIG = 2**30


@functools.lru_cache(maxsize=None)
def _build_gather(batch, num_rows):
    nbkt = (num_rows + 127) // 128
    last_bkt = nbkt - 1
    tmax = (nbkt - 1) // _NW + 1
    nrows_out = (batch + _NW + 7) // 8 * 8
    mesh = plsc.VectorSubcoreMesh(core_axis_name="c", subcore_axis_name="s")

    @functools.partial(
        pl.kernel,
        mesh=mesh,
        compiler_params=pltpu.CompilerParams(
            needs_layout_passes=False, disable_bounds_checks=True),
        out_type=(jax.ShapeDtypeStruct((nrows_out, 128), jnp.float32),
                  jax.ShapeDtypeStruct((nrows_out, 128), jnp.float32)),
        scratch_types=[
            pltpu.VMEM((batch,), jnp.int32),
            pltpu.VMEM((256, _CAP), jnp.int32),
            pltpu.VMEM((256, ), jnp.int32),
            pltpu.VMEM((batch,), jnp.int32),
            pltpu.VMEM((_RB, _D, 128), jnp.float32),
            pltpu.VMEM((_OB, 128), jnp.float32),
            pltpu.VMEM((_OB,), jnp.int32),
            pltpu.SemaphoreType.DMA((_RB,)),
            pltpu.SemaphoreType.DMA,
        ],
    )
    def _k(uidx_hbm, iidx_hbm, utab_t, itab_t, gu_hbm, gi_hbm,
           idx_v, ml_v, cnt_v, spill_v, ring, outbuf, bidx_v, rsems, fsem):
        wid = lax.axis_index("s") * _NC + lax.axis_index("c")
        dummy = batch + wid
        lane16 = lax.iota(jnp.int32, _L)
        zero16 = jnp.zeros((_L,), jnp.int32)

        def reset_bidx():
            for o in range(0, _OB, _L):
                bidx_v[pl.ds(o, _L)] = jnp.full((_L,), dummy, jnp.int32)

        def flush(g_hbm):
            pltpu.async_copy(outbuf, g_hbm.at[bidx_v], fsem).wait()
            reset_bidx()

        def get_scalar(ref, e):
            chunk = ref[pl.ds((e // _L) * _L, _L)]
            return jnp.sum(jnp.where(lane16 == (e % _L), chunk, 0))

        def fetch(tab, t, slot):
            j = lax.min(wid + _NW * t, last_bkt)
            start = pl.multiple_of(j * 128, 128)
            pltpu.async_copy(tab.at[:, pl.ds(start, 128)],
                             ring.at[slot], rsems.at[slot])

        def drain(tab, slot):
            pltpu.make_async_copy(
                tab.at[:, pl.ds(0, 128)], ring.at[slot], rsems.at[slot]
            ).wait()

        def do_table(idx_hbm, tab, g_hbm):
            pltpu.sync_copy(idx_hbm, idx_v)
            reset_bidx()
            for o in range(0, 256, _L):
                cnt_v[pl.ds(o, _L)] = zero16

            def scan_body(c, scnt):
                u = idx_v[pl.ds(c * _L, _L)]
                j = u >> 7
                own = (j & (_NW - 1)) == wid
                t = jnp.where(own, j >> 5, 255)
                occ, is_last = plsc.scan_count(t, mask=own)
                base = plsc.load_gather(cnt_v, [t])
                slot = base + occ - 1
                pack = (t << 21) | ((u & 127) << 14) | (c * _L + lane16)
                main = own & (slot < _CAP)
                plsc.store_scatter(
                    ml_v, [t, jnp.where(main, slot, 0)], pack, mask=main)
                spm = own & (slot >= _CAP)
                spmi = spm.astype(jnp.int32)
                spos = scnt + plsc.cumsum(spmi) - 1
                plsc.store_scatter(spill_v, [jnp.where(spm, spos, 0)],
                                   pack, mask=spm)
                plsc.addupdate_scatter(cnt_v, [t], occ,
                                       mask=own & is_last)
                return scnt + jnp.sum(spmi)

            scnt = lax.fori_loop(0, batch // _L, scan_body, 0)

            for s in range(_RB - 1):
                fetch(tab, s, s)

            def bucket_body(t, k):
                slot = lax.rem(t, _RB)
                fetch(tab, t + (_RB - 1), lax.rem(t + _RB - 1, _RB))
                drain(tab, slot)
                c = lax.min(get_scalar(cnt_v, t), _CAP)
                do_flush = k >= _OB - _CAP

                @pl.when(do_flush)
                def _():
                    flush(g_hbm)

                k = jnp.where(do_flush, 0, k)

                @pl.when(c > 0)
                def _(k=k):
                    pk = ml_v[t, pl.ds(0, _CAP)]
                    lane = (pk >> 14) & 127
                    b = pk & ((1 << 14) - 1)
                    m = lane16 < c
                    pos = k + lane16
                    slotv = jnp.full((_L,), slot, jnp.int32)
                    for d in range(_D):
                        vals = plsc.load_gather(
                            ring, [slotv, jnp.full((_L,), d, jnp.int32), lane])
                        plsc.store_scatter(
                            outbuf, [jnp.where(m, pos, 0),
                                     jnp.full((_L,), d, jnp.int32)],
                            vals, mask=m)
                    plsc.store_scatter(bidx_v, [jnp.where(m, pos, 0)],
                                       b, mask=m)
                return k + c

            k = lax.fori_loop(0, tmax - (_RB - 1), bucket_body, 0)
            for s in range(_RB - 1):
                t = tmax - (_RB - 1) + s
                slot = t % _RB
                drain(tab, slot)
                c = lax.min(get_scalar(cnt_v, t), _CAP)
                do_flush = k >= _OB - _CAP

                @pl.when(do_flush)
                def _():
                    flush(g_hbm)

                k = jnp.where(do_flush, 0, k)

                @pl.when(c > 0)
                def _(k=k, t=t, slot=slot):
                    pk = ml_v[t, pl.ds(0, _CAP)]
                    lane = (pk >> 14) & 127
                    b = pk & ((1 << 14) - 1)
                    m = lane16 < c
                    pos = k + lane16
                    slotv = jnp.full((_L,), slot, jnp.int32)
                    for d in range(_D):
                        vals = plsc.load_gather(
                            ring, [slotv, jnp.full((_L,), d, jnp.int32), lane])
                        plsc.store_scatter(
                            outbuf, [jnp.where(m, pos, 0),
                                     jnp.full((_L,), d, jnp.int32)],
                            vals, mask=m)
                    plsc.store_scatter(bidx_v, [jnp.where(m, pos, 0)],
                                       b, mask=m)
                k = k + c

            def spill_body(e, k):
                pk = get_scalar(spill_v, e)
                t = pk >> 21
                lane = (pk >> 14) & 127
                b = pk & ((1 << 14) - 1)
                fetch(tab, t, 0)
                drain(tab, 0)
                do_flush = k >= _OB - 1

                @pl.when(do_flush)
                def _():
                    flush(g_hbm)

                k = jnp.where(do_flush, 0, k)
                lo = plsc.load_gather(
                    ring, [zero16, lane16, jnp.full((_L,), lane, jnp.int32)])
                hi = plsc.load_gather(
                    ring, [zero16, lane16 + _L,
                           jnp.full((_L,), lane, jnp.int32)])
                kv = jnp.full((_L,), k, jnp.int32)
                plsc.store_scatter(outbuf, [kv, lane16], lo)
                plsc.store_scatter(outbuf, [kv, lane16 + _L], hi)
                plsc.store_scatter(bidx_v, [kv],
                                   jnp.full((_L,), b, jnp.int32),
                                   mask=lane16 == 0)
                return k + 1

            k = lax.fori_loop(0, scnt, spill_body, k)
            flush(g_hbm)

        do_table(uidx_hbm, utab_t, gu_hbm)
        do_table(iidx_hbm, itab_t, gi_hbm)

    return _k, nrows_out


@functools.lru_cache(maxsize=None)
def _build_dot(batch, nrows_out):
    blk = 512

    def body(u_ref, i_ref, o_ref):
        lane = lax.broadcasted_iota(jnp.int32, (blk, 128), 1)
        p = jnp.where(lane < _D, u_ref[...] * i_ref[...], 0.0)
        o_ref[...] = jnp.sum(p, axis=1)

    return pl.pallas_call(
        body,
        grid=(batch // blk,),
        in_specs=[pl.BlockSpec((blk, 128), lambda i: (i, 0)),
                  pl.BlockSpec((blk, 128), lambda i: (i, 0))],
        out_specs=pl.BlockSpec((blk,), lambda i: (i,)),
        out_shape=jax.ShapeDtypeStruct((batch,), jnp.float32),
    )


def kernel(user_indices, item_indices, user_table, item_table):
    batch = user_indices.shape[0]
    gather, nrows_out = _build_gather(batch, user_table.shape[0])
    gu, gi = gather(user_indices.astype(jnp.int32),
                    item_indices.astype(jnp.int32),
                    user_table.T, item_table.T)
    return _build_dot(batch, nrows_out)(gu, gi)


# speedup vs baseline: 1.1112x; 1.1112x over previous
import functools

import jax
import jax.numpy as jnp
from jax import lax
from jax.experimental import pallas as pl
from jax.experimental.pallas import tpu as pltpu
from jax.experimental.pallas import tpu_sc as plsc

_D = 32
_L = 16
_NC = 2
_NS = 16
_NW = _NC * _NS
_NBUF = 12


@functools.lru_cache(maxsize=None)
def _build(batch, num_rows):
    bpw = batch // _NW
    last_win = ((num_rows - 1) // 128) * 128
    mesh = plsc.VectorSubcoreMesh(core_axis_name="c", subcore_axis_name="s")

    @functools.partial(
        pl.kernel,
        mesh=mesh,
        compiler_params=pltpu.CompilerParams(
            needs_layout_passes=False, disable_bounds_checks=True),
        out_type=jax.ShapeDtypeStruct((batch,), jnp.float32),
        scratch_types=[
            pltpu.VMEM((bpw,), jnp.int32),
            pltpu.VMEM((bpw,), jnp.int32),
            pltpu.VMEM((bpw,), jnp.int32),
            pltpu.VMEM((bpw,), jnp.int32),
            pltpu.VMEM((_NBUF, _D, 128), jnp.float32),
            pltpu.VMEM((_NBUF, _D, 128), jnp.float32),
            pltpu.VMEM((bpw,), jnp.float32),
            pltpu.SemaphoreType.DMA((_NBUF,)),
        ],
    )
    def _k(uidx_hbm, iidx_hbm, utab_t, itab_t, out_hbm,
           uws_v, iws_v, ul_v, il_v, uwin, iwin, out_v, sems):
        wid = lax.axis_index("s") * _NC + lax.axis_index("c")
        base = wid * bpw
        lane16 = lax.iota(jnp.int32, _L)

        pltpu.sync_copy(uidx_hbm.at[pl.ds(base, bpw)], ul_v)
        pltpu.sync_copy(iidx_hbm.at[pl.ds(base, bpw)], il_v)
        for c in range(bpw // _L):
            u = ul_v[pl.ds(c * _L, _L)]
            ws = jnp.minimum((u >> 7) << 7, last_win)
            uws_v[pl.ds(c * _L, _L)] = ws
            ul_v[pl.ds(c * _L, _L)] = u - ws
            v = il_v[pl.ds(c * _L, _L)]
            ws = jnp.minimum((v >> 7) << 7, last_win)
            iws_v[pl.ds(c * _L, _L)] = ws
            il_v[pl.ds(c * _L, _L)] = v - ws

        def get_scalar(ref, b):
            chunk = ref[pl.ds((b // _L) * _L, _L)]
            return jnp.sum(jnp.where(lane16 == (b % _L), chunk, 0))

        def fetch(b, slot):
            us = pl.multiple_of(get_scalar(uws_v, b), 128)
            vs = pl.multiple_of(get_scalar(iws_v, b), 128)
            pltpu.async_copy(
                utab_t.at[:, pl.ds(us, 128)], uwin.at[slot], sems.at[slot])
            pltpu.async_copy(
                itab_t.at[:, pl.ds(vs, 128)], iwin.at[slot], sems.at[slot])

        def drain(slot):
            pltpu.make_async_copy(
                utab_t.at[:, pl.ds(0, 128)], uwin.at[slot], sems.at[slot]
            ).wait()
            pltpu.make_async_copy(
                itab_t.at[:, pl.ds(0, 128)], iwin.at[slot], sems.at[slot]
            ).wait()

        def compute(b, slot):
            ul = jnp.full((_L,), get_scalar(ul_v, b), jnp.int32)
            vl = jnp.full((_L,), get_scalar(il_v, b), jnp.int32)
            sl = jnp.full((_L,), slot, jnp.int32)
            u_lo = plsc.load_gather(uwin, [sl, lane16, ul])
            u_hi = plsc.load_gather(uwin, [sl, lane16 + _L, ul])
            i_lo = plsc.load_gather(iwin, [sl, lane16, vl])
            i_hi = plsc.load_gather(iwin, [sl, lane16 + _L, vl])
            s = u_lo * i_lo + u_hi * i_hi
            tot = jnp.sum(s)
            plsc.store_scatter(
                out_v, [jnp.full((_L,), b, jnp.int32)],
                jnp.full((_L,), tot, jnp.float32),
                mask=lane16 == 0)

        for s in range(_NBUF - 1):
            fetch(s, s)

        def body(b, carry):
            slot = lax.rem(b, _NBUF)
            fetch(b + (_NBUF - 1), lax.rem(b + _NBUF - 1, _NBUF))
            drain(slot)
            compute(b, slot)
            return carry

        lax.fori_loop(0, bpw - (_NBUF - 1), body, 0)

        for t in range(_NBUF - 1):
            b = bpw - (_NBUF - 1) + t
            drain(b % _NBUF)
            compute(b, b % _NBUF)

        pltpu.sync_copy(out_v, out_hbm.at[pl.ds(base, bpw)])

    return _k


def kernel(user_indices, item_indices, user_table, item_table):
    batch = user_indices.shape[0]
    k = _build(batch, user_table.shape[0])
    return k(user_indices.astype(jnp.int32), item_indices.astype(jnp.int32),
             user_table.T, item_table.T)
